# Initial kernel scaffold; baseline (speedup 1.0000x reference)
import jax
import jax.numpy as jnp
from jax.experimental import pallas as pl


def kernel(x, edge_index, W1, b1, a1, W2, b2, a2):
    raise NotImplementedError("write your pallas kernel here")


# speedup vs baseline: 9.8263x; 9.8263x over previous
import jax
import jax.numpy as jnp
from jax import lax
from jax.experimental import pallas as pl
from jax.experimental.pallas import tpu as pltpu
from jax.experimental.pallas import tpu_sc as plsc

NC = 2
NS = 16
L = 16
CH = 80
DH = 64
ZR = 125
_R = 1000

_SC_PARAMS = pltpu.CompilerParams(
    needs_layout_passes=False, use_tc_tiling_on_sc=False)


def _sc_degree(dst, n, e):
  nw = NC * NS
  epw = e // nw
  nb = n // _R
  mesh = plsc.VectorSubcoreMesh(core_axis_name="c", subcore_axis_name="s")

  def body(dst_hbm, out_hbm, didx, deg):
    c = lax.axis_index("c")
    s = lax.axis_index("s")
    wid = s * NC + c
    pltpu.sync_copy(dst_hbm.at[pl.ds(wid * epw, epw)], didx)
    zeros = jnp.zeros((L,), jnp.float32)

    def zb(i, carry):
      deg[pl.ds(i * L, L)] = zeros
      return carry
    lax.fori_loop(0, n // L, zb, 0)

    ones = jnp.ones((L,), jnp.float32)

    def hb(i, carry):
      idx = didx[pl.ds(i * L, L)]
      plsc.addupdate_scatter(deg, [idx], ones)
      return carry
    lax.fori_loop(0, epw // L, hb, 0)
    for b in range(nb):
      pltpu.sync_copy(deg.at[pl.ds(b * _R, _R)], out_hbm.at[b].at[wid])

  f = pl.kernel(
      body,
      out_type=jax.ShapeDtypeStruct((nb, nw, _R), jnp.float32),
      mesh=mesh,
      compiler_params=_SC_PARAMS,
      scratch_types=[
          pltpu.VMEM((epw,), jnp.int32),
          pltpu.VMEM((n,), jnp.float32),
      ],
  )
  return f(dst)


def _sc_layer(u0_flat, src, dst, dinv2rep, ng, n, e):
  gc = ng // NC
  eps = e // NS
  nch = eps // CH
  rpt = n // NS
  mesh = plsc.VectorSubcoreMesh(core_axis_name="c", subcore_axis_name="s")

  def body(u0, src_h, dst_h, dv_h, s1, s2, s3, uw,
           sidx, didxall, didxb, rows0, rows1, zbuf, scbuf, dvv, acc,
           sem0, sem1):
    c = lax.axis_index("c")
    s = lax.axis_index("s")
    ebase = s * eps
    row0 = s * rpt
    pltpu.sync_copy(src_h.at[pl.ds(ebase, eps)], sidx)
    pltpu.sync_copy(dst_h.at[pl.ds(ebase, eps)], didxall)
    pltpu.sync_copy(dv_h.at[pl.ds(row0, rpt)], dvv)

    off = jnp.full((L,), c * gc * n, jnp.int32)

    def ob(i, carry):
      sidx[pl.ds(i * L, L)] = sidx[pl.ds(i * L, L)] + off
      return carry
    lax.fori_loop(0, eps // L, ob, 0)

    zeros = jnp.zeros((L,), jnp.float32)
    qpr = DH // L

    def zb(i, carry):
      zbuf[i // qpr, pl.ds((i % qpr) * L, L)] = zeros
      return carry
    lax.fori_loop(0, ZR * qpr, zb, 0)

    bump = jnp.full((L,), n, jnp.int32)

    def bumpidx(i, carry):
      sidx[pl.ds(i * L, L)] = sidx[pl.ds(i * L, L)] + bump
      return carry

    def gissue(uin, j, rbuf, sem):
      pltpu.async_copy(uin.at[sidx.at[pl.ds(j * CH, CH)]], rbuf, sem)

    def gwait(uin, j, rbuf, sem):
      pltpu.make_async_copy(uin.at[sidx.at[pl.ds(j * CH, CH)]], rbuf,
                            sem).wait()

    def scat(j, rbuf):
      for t in range(CH // L):
        didxb[pl.ds(t * L, L)] = didxall[pl.ds(j * CH + t * L, L)]
      pltpu.sync_copy(rbuf, acc.at[didxb], add=True)

    for k in range(3):
      uin = u0 if k == 0 else uw
      sout = (s1, s2, s3)[k]
      for p in range(gc):
        if p > 0:
          lax.fori_loop(0, eps // L, bumpidx, 0)
        g = c * gc + p
        for r in range(rpt // ZR):
          pltpu.sync_copy(zbuf, acc.at[pl.ds(row0 + r * ZR, ZR)])
        plsc.subcore_barrier()

        gissue(uin, 0, rows0, sem0)

        def eb(jj, carry, _uin=uin):
          j0 = 2 * jj
          j1 = 2 * jj + 1
          gissue(_uin, j1, rows1, sem1)
          gwait(_uin, j0, rows0, sem0)
          scat(j0, rows0)

          @pl.when(j1 + 1 < nch)
          def _():
            gissue(_uin, j1 + 1, rows0, sem0)
          gwait(_uin, j1, rows1, sem1)
          scat(j1, rows1)
          return carry
        lax.fori_loop(0, nch // 2, eb, 0)
        plsc.subcore_barrier()

        for r in range(rpt // ZR):
          rr0 = row0 + r * ZR
          pltpu.sync_copy(acc.at[pl.ds(rr0, ZR)],
                          sout.at[pl.ds(g * n + rr0, ZR)])
        if k < 2:
          for r in range(rpt // ZR):
            rr0 = row0 + r * ZR
            pltpu.sync_copy(acc.at[pl.ds(rr0, ZR)], scbuf)

            def sb(q, carry, _r=r):
              rr = q // qpr
              qq = q % qpr
              dvrow = dvv[_r * ZR + rr]
              scbuf[rr, pl.ds(qq * L, L)] = (
                  scbuf[rr, pl.ds(qq * L, L)] * dvrow)
              return carry
            lax.fori_loop(0, ZR * qpr, sb, 0)
            pltpu.sync_copy(scbuf, uw.at[pl.ds(g * n + rr0, ZR)])
        plsc.subcore_barrier()
      if gc > 1:
        off2 = jnp.full((L,), (gc - 1) * n, jnp.int32)

        def rewind(i, carry):
          sidx[pl.ds(i * L, L)] = sidx[pl.ds(i * L, L)] - off2
          return carry
        lax.fori_loop(0, eps // L, rewind, 0)

  f = pl.kernel(
      body,
      out_type=[jax.ShapeDtypeStruct((ng * n, DH), jnp.float32)] * 4,
      mesh=mesh,
      compiler_params=_SC_PARAMS,
      scratch_types=[
          pltpu.VMEM((eps,), jnp.int32),
          pltpu.VMEM((eps,), jnp.int32),
          pltpu.VMEM((CH,), jnp.int32),
          pltpu.VMEM((CH, DH), jnp.float32),
          pltpu.VMEM((CH, DH), jnp.float32),
          pltpu.VMEM((ZR, DH), jnp.float32),
          pltpu.VMEM((ZR, DH), jnp.float32),
          pltpu.VMEM((rpt, L), jnp.float32),
          pltpu.VMEM_SHARED((n, DH), jnp.float32),
          pltpu.SemaphoreType.DMA,
          pltpu.SemaphoreType.DMA,
      ],
  )
  return f(u0_flat, src, dst, dinv2rep)


def _dinv_block(deg_ref):
  deg = jnp.sum(deg_ref[0], axis=0)
  return jnp.where(deg > 0, lax.rsqrt(jnp.maximum(deg, 1e-12)), 0.0)


def _deg_spec():
  return pl.BlockSpec((1, NC * NS, _R), lambda i: (i, 0, 0))


def _split_groups(u, u_ref, ng):
  for g in range(ng):
    u_ref[g] = u[:, g * DH:(g + 1) * DH]


def _cat_groups(sref, ng):
  return jnp.concatenate([sref[g] for g in range(ng)], axis=1)


def _tc_prep(deg_p, x, W1, n, d_in, hid):
  g = n // _R
  ng = d_in // DH

  def body(deg_ref, x_ref, w_ref, dv_ref, u0_ref, acc_ref):
    dinv = _dinv_block(deg_ref)
    xb = x_ref[...]
    _split_groups(xb * dinv[:, None], u0_ref, ng)
    dv_ref[...] = jnp.broadcast_to((dinv * dinv)[:, None], (_R, L))
    acc_ref[...] = jnp.dot(xb, w_ref[0], preferred_element_type=jnp.float32)

  return pl.pallas_call(
      body,
      grid=(g,),
      in_specs=[
          _deg_spec(),
          pl.BlockSpec((_R, d_in), lambda i: (i, 0)),
          pl.BlockSpec(W1.shape, lambda i: (0, 0, 0)),
      ],
      out_specs=[
          pl.BlockSpec((_R, L), lambda i: (i, 0)),
          pl.BlockSpec((ng, _R, DH), lambda i: (0, i, 0)),
          pl.BlockSpec((_R, hid), lambda i: (i, 0)),
      ],
      out_shape=[
          jax.ShapeDtypeStruct((n, L), jnp.float32),
          jax.ShapeDtypeStruct((ng, n, DH), jnp.float32),
          jax.ShapeDtypeStruct((n, hid), jnp.float32),
      ],
  )(deg_p, x, W1)


def _tc_mid(deg_p, acc0, s1, s2, s3, W1, b1, a1, W2, n, d_in, hid):
  g = n // _R
  ng1 = d_in // DH
  ng2 = hid // DH

  def body(deg_ref, acc_ref, s1_ref, s2_ref, s3_ref, w1_ref, b1_ref, a1_ref,
           w2_ref, u0_ref, acc2_ref):
    dinv = _dinv_block(deg_ref)
    h = acc_ref[...]
    for k, sref in enumerate((s1_ref, s2_ref, s3_ref)):
      sk = _cat_groups(sref, ng1) * dinv[:, None]
      h = h + jnp.dot(sk, w1_ref[k + 1], preferred_element_type=jnp.float32)
    h = h + b1_ref[...]
    a = a1_ref[0, 0]
    h = jnp.where(h >= 0, h, a * h)
    _split_groups(h * dinv[:, None], u0_ref, ng2)
    acc2_ref[...] = jnp.dot(h, w2_ref[0], preferred_element_type=jnp.float32)

  sspec = pl.BlockSpec((ng1, _R, DH), lambda i: (0, i, 0))
  return pl.pallas_call(
      body,
      grid=(g,),
      in_specs=[
          _deg_spec(),
          pl.BlockSpec((_R, hid), lambda i: (i, 0)),
          sspec, sspec, sspec,
          pl.BlockSpec(W1.shape, lambda i: (0, 0, 0)),
          pl.BlockSpec((1, hid), lambda i: (0, 0)),
          pl.BlockSpec((1, 1), lambda i: (0, 0)),
          pl.BlockSpec(W2.shape, lambda i: (0, 0, 0)),
      ],
      out_specs=[
          pl.BlockSpec((ng2, _R, DH), lambda i: (0, i, 0)),
          pl.BlockSpec((_R, W2.shape[2]), lambda i: (i, 0)),
      ],
      out_shape=[
          jax.ShapeDtypeStruct((ng2, n, DH), jnp.float32),
          jax.ShapeDtypeStruct((n, W2.shape[2]), jnp.float32),
      ],
  )(deg_p, acc0, s1, s2, s3, W1, b1, a1, W2)


def _tc_final(deg_p, acc2, t1, t2, t3, W2, b2, a2, n, hid, d_out):
  g = n // _R
  ng = hid // DH

  def body(deg_ref, acc_ref, s1_ref, s2_ref, s3_ref, w_ref, b_ref, a_ref,
           y_ref):
    dinv = _dinv_block(deg_ref)
    h = acc_ref[...]
    for k, sref in enumerate((s1_ref, s2_ref, s3_ref)):
      sk = _cat_groups(sref, ng) * dinv[:, None]
      h = h + jnp.dot(sk, w_ref[k + 1], preferred_element_type=jnp.float32)
    h = h + b_ref[...]
    a = a_ref[0, 0]
    y_ref[...] = jnp.where(h >= 0, h, a * h)

  sspec = pl.BlockSpec((ng, _R, DH), lambda i: (0, i, 0))
  return pl.pallas_call(
      body,
      grid=(g,),
      in_specs=[
          _deg_spec(),
          pl.BlockSpec((_R, d_out), lambda i: (i, 0)),
          sspec, sspec, sspec,
          pl.BlockSpec(W2.shape, lambda i: (0, 0, 0)),
          pl.BlockSpec((1, d_out), lambda i: (0, 0)),
          pl.BlockSpec((1, 1), lambda i: (0, 0)),
      ],
      out_specs=pl.BlockSpec((_R, d_out), lambda i: (i, 0)),
      out_shape=jax.ShapeDtypeStruct((n, d_out), jnp.float32),
  )(deg_p, acc2, t1, t2, t3, W2, b2, a2)


def kernel(x, edge_index, W1, b1, a1, W2, b2, a2):
  n, d_in = x.shape
  e = edge_index.shape[1]
  hid = W1.shape[2]
  d_out = W2.shape[2]
  src = edge_index[0]
  dst = edge_index[1]
  b1r = b1.reshape(1, hid)
  a1r = a1.reshape(1, 1)
  b2r = b2.reshape(1, d_out)
  a2r = a2.reshape(1, 1)
  ng1 = d_in // DH
  ng2 = hid // DH

  deg_p = _sc_degree(dst, n, e)
  dinv2rep, u0, acc0 = _tc_prep(deg_p, x, W1, n, d_in, hid)
  s1, s2, s3, _ = _sc_layer(u0.reshape(ng1 * n, DH), src, dst, dinv2rep,
                            ng1, n, e)
  rs1 = lambda v, ng: v.reshape(ng, n, DH)
  u0b, acc2 = _tc_mid(deg_p, acc0, rs1(s1, ng1), rs1(s2, ng1), rs1(s3, ng1),
                      W1, b1r, a1r, W2, n, d_in, hid)
  t1, t2, t3, _ = _sc_layer(u0b.reshape(ng2 * n, DH), src, dst, dinv2rep,
                            ng2, n, e)
  return _tc_final(deg_p, acc2, rs1(t1, ng2), rs1(t2, ng2), rs1(t3, ng2),
                   W2, b2r, a2r, n, hid, d_out)
